# baseline (speedup 1.0000x reference)
import functools

import jax
import jax.numpy as jnp
from jax import lax
from jax.experimental import pallas as pl
from jax.experimental.pallas import tpu as pltpu
from jax.experimental.pallas import tpu_sc as plsc

_NC = 2
_NS = 16
_NW = _NC * _NS
_CHUNK = 32


def _embed(ids, wte, wpe, *, n, d, seq, bpw, nch):
    mesh = plsc.VectorSubcoreMesh(
        core_axis_name="c", subcore_axis_name="s",
        num_cores=_NC, num_subcores=_NS)

    @functools.partial(
        pl.kernel,
        out_type=jax.ShapeDtypeStruct((n, d), jnp.float32),
        mesh=mesh,
        scratch_types=[
            pltpu.VMEM((nch, _CHUNK), jnp.int32),
            pltpu.VMEM((_CHUNK, d), jnp.float32),
            pltpu.VMEM((_CHUNK, d), jnp.float32),
            pltpu.SemaphoreType.DMA,
            pltpu.SemaphoreType.DMA,
        ],
    )
    def run(ids_hbm, wte_hbm, wpe_hbm, out_hbm, idx_v, buf_v, wpe_v, sem, sem2):
        wid = lax.axis_index("s") * _NC + lax.axis_index("c")
        base = wid * bpw
        s0 = base % seq
        pltpu.sync_copy(ids_hbm.at[wid], idx_v)

        def chunk_body(c, carry):
            cp_wte = pltpu.async_copy(wte_hbm.at[idx_v.at[c]], buf_v, sem)
            cp_wpe = pltpu.async_copy(
                wpe_hbm.at[pl.ds(s0 + c * _CHUNK, _CHUNK)], wpe_v, sem2)
            cp_wte.wait()
            cp_wpe.wait()

            def row_body(r, rcarry):
                for k in range(d // 16):
                    sl = pl.ds(k * 16, 16)
                    plsc.addupdate(buf_v.at[r, sl], wpe_v[r, sl])
                return rcarry

            lax.fori_loop(0, _CHUNK, row_body, 0)
            pltpu.sync_copy(buf_v, out_hbm.at[pl.ds(base + c * _CHUNK, _CHUNK)])
            return carry

        lax.fori_loop(0, nch, chunk_body, 0)

    return run(ids, wte, wpe)


def kernel(input_ids, wte, wpe):
    b, seq = input_ids.shape
    d = wte.shape[1]
    n = b * seq
    bpw = n // _NW
    nch = bpw // _CHUNK
    ids = input_ids.reshape(_NW, nch, _CHUNK).astype(jnp.int32)
    out = _embed(ids, wte, wpe, n=n, d=d, seq=seq, bpw=bpw, nch=nch)
    return out.reshape(b, seq, d)


# speedup vs baseline: 1.4427x; 1.4427x over previous
import functools

import jax
import jax.numpy as jnp
from jax import lax
from jax.experimental import pallas as pl
from jax.experimental.pallas import tpu as pltpu
from jax.experimental.pallas import tpu_sc as plsc

_NC = 2
_NS = 16
_NW = _NC * _NS
_P = 16
_NB = 4


def _embed(ids, wte, wpe, *, n, d, seq, nbatch):
    span = seq // _NW
    npsub = span // _P
    nsteps = npsub * nbatch

    mesh = plsc.VectorSubcoreMesh(
        core_axis_name="c", subcore_axis_name="s",
        num_cores=_NC, num_subcores=_NS)

    @functools.partial(
        pl.kernel,
        out_type=jax.ShapeDtypeStruct((n, d), jnp.float32),
        mesh=mesh,
        scratch_types=[
            pltpu.VMEM((npsub, nbatch, _P), jnp.int32),
            pltpu.VMEM((_NB, _P, d), jnp.float32),
            pltpu.VMEM((2, _P, d), jnp.float32),
            pltpu.SemaphoreType.DMA((_NB,)),
            pltpu.SemaphoreType.DMA((_NB,)),
            pltpu.SemaphoreType.DMA((2,)),
        ],
    )
    def run(ids_hbm, wte_hbm, wpe_hbm, out_hbm, idx_v, gbuf, wbuf, gsem, ssem, wsem):
        wid = lax.axis_index("s") * _NC + lax.axis_index("c")
        s0 = wid * span

        pltpu.sync_copy(ids_hbm.at[wid], idx_v)

        def g_copy(i):
            p, b, be = i // nbatch, i % nbatch, i % _NB
            return pltpu.make_async_copy(
                wte_hbm.at[idx_v.at[p, b]], gbuf.at[be], gsem.at[be])

        def s_copy(i):
            p, b, be = i // nbatch, i % nbatch, i % _NB
            row0 = b * seq + s0 + p * _P
            return pltpu.make_async_copy(
                gbuf.at[be], out_hbm.at[pl.ds(row0, _P)], ssem.at[be])

        def w_copy(pp):
            return pltpu.make_async_copy(
                wpe_hbm.at[pl.ds(s0 + pp * _P, _P)], wbuf.at[pp % 2],
                wsem.at[pp % 2])

        w_copy(0).start()
        w_copy(1).start()
        g_copy(0).start()
        g_copy(1).start()

        def step(i, carry):
            p = i // nbatch
            b = i % nbatch
            be = i % _NB

            @pl.when(i >= 2)
            def _():
                s_copy(i - 2).wait()

            @pl.when(i + 2 < nsteps)
            def _():
                g_copy(i + 2).start()

            g_copy(i).wait()

            @pl.when(b == 0)
            def _():
                w_copy(p).wait()

            wsel = p % 2

            def row_body(r, rcarry):
                for k in range(d // 16):
                    sl = pl.ds(k * 16, 16)
                    plsc.addupdate(gbuf.at[be, r, sl], wbuf[wsel, r, sl])
                return rcarry

            lax.fori_loop(0, _P, row_body, 0)
            s_copy(i).start()

            @pl.when(jnp.logical_and(b == nbatch - 1, p + 2 < npsub))
            def _():
                w_copy(p + 2).start()

            return carry

        lax.fori_loop(0, nsteps, step, 0)
        s_copy(nsteps - 2).wait()
        s_copy(nsteps - 1).wait()

    return run(ids, wte, wpe)


def kernel(input_ids, wte, wpe):
    nbatch, seq = input_ids.shape
    d = wte.shape[1]
    n = nbatch * seq
    span = seq // _NW
    npsub = span // _P
    ids = (input_ids.astype(jnp.int32)
           .reshape(nbatch, _NW, npsub, _P)
           .transpose(1, 2, 0, 3))
    out = _embed(ids, wte, wpe, n=n, d=d, seq=seq, nbatch=nbatch)
    return out.reshape(nbatch, seq, d)


# speedup vs baseline: 2.6592x; 1.8433x over previous
import functools

import jax
import jax.numpy as jnp
from jax import lax
from jax.experimental import pallas as pl
from jax.experimental.pallas import tpu as pltpu
from jax.experimental.pallas import tpu_sc as plsc

_NC = 2
_NS = 16
_NW = _NC * _NS
_P = 16
_NB = 4


def _embed(ids, wte, wpe, *, n, d, seq, nbatch):
    span = seq // _NW
    npsub = span // _P
    nsteps = npsub * nbatch

    mesh = plsc.VectorSubcoreMesh(
        core_axis_name="c", subcore_axis_name="s",
        num_cores=_NC, num_subcores=_NS)

    @functools.partial(
        pl.kernel,
        out_type=jax.ShapeDtypeStruct((n, d), jnp.float32),
        mesh=mesh,
        scratch_types=[
            pltpu.VMEM((npsub, nbatch, _P), jnp.int32),
            pltpu.VMEM((_NB, _P, d), jnp.float32),
            pltpu.VMEM((2, _P, d), jnp.float32),
            pltpu.SemaphoreType.DMA((_NB,)),
            pltpu.SemaphoreType.DMA((_NB,)),
            pltpu.SemaphoreType.DMA((2,)),
        ],
    )
    def run(ids_hbm, wte_hbm, wpe_hbm, out_hbm, idx_v, gbuf, wbuf, gsem, ssem, wsem):
        wid = lax.axis_index("s") * _NC + lax.axis_index("c")
        s0 = wid * span

        pltpu.sync_copy(ids_hbm.at[wid], idx_v)

        def g_copy(i):
            p, b, be = i // nbatch, i % nbatch, i % _NB
            return pltpu.make_async_copy(
                wte_hbm.at[idx_v.at[p, b]], gbuf.at[be], gsem.at[be])

        def s_copy(i):
            p, b, be = i // nbatch, i % nbatch, i % _NB
            row0 = b * seq + s0 + p * _P
            return pltpu.make_async_copy(
                gbuf.at[be], out_hbm.at[pl.ds(row0, _P)], ssem.at[be])

        def w_copy(pp):
            return pltpu.make_async_copy(
                wpe_hbm.at[pl.ds(s0 + pp * _P, _P)], wbuf.at[pp % 2],
                wsem.at[pp % 2])

        w_copy(0).start()
        w_copy(1).start()
        g_copy(0).start()
        g_copy(1).start()

        def step(i, carry):
            p = i // nbatch
            b = i % nbatch
            be = i % _NB

            @pl.when(i >= 2)
            def _():
                s_copy(i - 2).wait()

            @pl.when(i + 2 < nsteps)
            def _():
                g_copy(i + 2).start()

            g_copy(i).wait()

            @pl.when(b == 0)
            def _():
                w_copy(p).wait()

            wsel = p % 2
            nvec = d // 16

            @plsc.parallel_loop(0, _P * nvec, 1, unroll=8)
            def add_body(v):
                r = v // nvec
                sl = pl.ds((v % nvec) * 16, 16)
                plsc.addupdate(gbuf.at[be, r, sl], wbuf[wsel, r, sl])
            s_copy(i).start()

            @pl.when(jnp.logical_and(b == nbatch - 1, p + 2 < npsub))
            def _():
                w_copy(p + 2).start()

            return carry

        lax.fori_loop(0, nsteps, step, 0)
        s_copy(nsteps - 2).wait()
        s_copy(nsteps - 1).wait()

    return run(ids, wte, wpe)


def kernel(input_ids, wte, wpe):
    nbatch, seq = input_ids.shape
    d = wte.shape[1]
    n = nbatch * seq
    span = seq // _NW
    npsub = span // _P
    ids = (input_ids.astype(jnp.int32)
           .reshape(nbatch, _NW, npsub, _P)
           .transpose(1, 2, 0, 3))
    out = _embed(ids, wte, wpe, n=n, d=d, seq=seq, nbatch=nbatch)
    return out.reshape(nbatch, seq, d)


# speedup vs baseline: 2.6834x; 1.0091x over previous
import functools

import jax
import jax.numpy as jnp
from jax import lax
from jax.experimental import pallas as pl
from jax.experimental.pallas import tpu as pltpu
from jax.experimental.pallas import tpu_sc as plsc

_NC = 2
_NS = 16
_NW = _NC * _NS
_P = 16
_NB = 4


def _embed(ids, wte, wpe, *, n, d, seq, nbatch):
    span = seq // _NW
    npsub = span // _P
    nsteps = npsub * nbatch

    mesh = plsc.VectorSubcoreMesh(
        core_axis_name="c", subcore_axis_name="s",
        num_cores=_NC, num_subcores=_NS)

    @functools.partial(
        pl.kernel,
        out_type=jax.ShapeDtypeStruct((n, d), jnp.float32),
        mesh=mesh,
        scratch_types=[
            pltpu.VMEM((nbatch, span), jnp.int32),
            pltpu.VMEM((_NB, _P, d), jnp.float32),
            pltpu.VMEM((2, _P, d), jnp.float32),
            pltpu.SemaphoreType.DMA((_NB,)),
            pltpu.SemaphoreType.DMA((_NB,)),
            pltpu.SemaphoreType.DMA((2,)),
        ],
    )
    def run(ids_hbm, wte_hbm, wpe_hbm, out_hbm, idx_v, gbuf, wbuf, gsem, ssem, wsem):
        wid = lax.axis_index("s") * _NC + lax.axis_index("c")
        s0 = wid * span

        pltpu.sync_copy(ids_hbm.at[:, pl.ds(s0, span)], idx_v)

        def g_copy(i):
            p, b, be = i // nbatch, i % nbatch, i % _NB
            return pltpu.make_async_copy(
                wte_hbm.at[idx_v.at[b, pl.ds(p * _P, _P)]],
                gbuf.at[be], gsem.at[be])

        def s_copy(i):
            p, b, be = i // nbatch, i % nbatch, i % _NB
            row0 = b * seq + s0 + p * _P
            return pltpu.make_async_copy(
                gbuf.at[be], out_hbm.at[pl.ds(row0, _P)], ssem.at[be])

        def w_copy(pp):
            return pltpu.make_async_copy(
                wpe_hbm.at[pl.ds(s0 + pp * _P, _P)], wbuf.at[pp % 2],
                wsem.at[pp % 2])

        w_copy(0).start()
        w_copy(1).start()
        g_copy(0).start()
        g_copy(1).start()

        def step(i, carry):
            p = i // nbatch
            b = i % nbatch
            be = i % _NB

            @pl.when(i >= 2)
            def _():
                s_copy(i - 2).wait()

            @pl.when(i + 2 < nsteps)
            def _():
                g_copy(i + 2).start()

            g_copy(i).wait()

            @pl.when(b == 0)
            def _():
                w_copy(p).wait()

            wsel = p % 2
            nvec = d // 16

            @plsc.parallel_loop(0, _P * nvec, 1, unroll=16)
            def add_body(v):
                r = v // nvec
                sl = pl.ds((v % nvec) * 16, 16)
                plsc.addupdate(gbuf.at[be, r, sl], wbuf[wsel, r, sl])
            s_copy(i).start()

            @pl.when(jnp.logical_and(b == nbatch - 1, p + 2 < npsub))
            def _():
                w_copy(p + 2).start()

            return carry

        lax.fori_loop(0, nsteps, step, 0)
        s_copy(nsteps - 2).wait()
        s_copy(nsteps - 1).wait()

    return run(ids, wte, wpe)


def kernel(input_ids, wte, wpe):
    nbatch, seq = input_ids.shape
    d = wte.shape[1]
    n = nbatch * seq
    out = _embed(input_ids.astype(jnp.int32), wte, wpe,
                 n=n, d=d, seq=seq, nbatch=nbatch)
    return out.reshape(nbatch, seq, d)
